# baseline (speedup 1.0000x reference)
import jax
import jax.numpy as jnp
from jax.experimental import pallas as pl
from jax.experimental.pallas import tpu as pltpu


def _probe_body(idx_ref, w_ref, feat_ref, out_ref, bulk_sem, row_sem):
    B, S, _ = feat_ref.shape

    def issue(b, carry):
        pltpu.make_async_copy(
            feat_ref.at[b], out_ref.at[b, pl.ds(0, S), :], bulk_sem
        ).start()
        return carry

    jax.lax.fori_loop(0, B, issue, 0)

    def drain(b, carry):
        pltpu.make_async_copy(
            feat_ref.at[0], out_ref.at[0, pl.ds(0, S), :], bulk_sem
        ).wait()
        return carry

    jax.lax.fori_loop(0, B, drain, 0)


def kernel(feature, index_value, embedding_offset, W):
    B, S, D = feature.shape
    idx = (index_value - embedding_offset).astype(jnp.int32)
    return pl.pallas_call(
        _probe_body,
        in_specs=[
            pl.BlockSpec(memory_space=pltpu.SMEM),
            pl.BlockSpec(memory_space=pl.ANY),
            pl.BlockSpec(memory_space=pl.ANY),
        ],
        out_specs=pl.BlockSpec(memory_space=pl.ANY),
        out_shape=jax.ShapeDtypeStruct((B, S + 1, D), feature.dtype),
        scratch_shapes=[pltpu.SemaphoreType.DMA, pltpu.SemaphoreType.DMA],
    )(idx, W, feature)


# speedup vs baseline: 21.3782x; 21.3782x over previous
import jax
import jax.numpy as jnp
from jax.experimental import pallas as pl
from jax.experimental.pallas import tpu as pltpu

_BB = 4
_SC = 2


def _concat_body(idx_ref, w_ref, feat_ref, out_ref):
    g = pl.program_id(0)
    c = pl.program_id(1)
    chunk = feat_ref.shape[1]

    @pl.when(c == 0)
    def _emb_rows():
        for j in range(_BB):
            out_ref[j, 0, :] = w_ref[idx_ref[g * _BB + j], :]

    for cc in range(_SC):

        @pl.when(c == cc)
        def _store_chunk(cc=cc):
            for j in range(_BB):
                out_ref[j, pl.ds(1 + cc * chunk, chunk), :] = feat_ref[j]


def kernel(feature, index_value, embedding_offset, W):
    B, S, D = feature.shape
    idx = (index_value - embedding_offset).astype(jnp.int32)
    V = W.shape[0]
    grid_spec = pltpu.PrefetchScalarGridSpec(
        num_scalar_prefetch=1,
        grid=(B // _BB, _SC),
        in_specs=[
            pl.BlockSpec((V, D), lambda g, c, idx_ref: (0, 0)),
            pl.BlockSpec((_BB, S // _SC, D), lambda g, c, idx_ref: (g, c, 0)),
        ],
        out_specs=pl.BlockSpec((_BB, S + 1, D), lambda g, c, idx_ref: (g, 0, 0)),
    )
    return pl.pallas_call(
        _concat_body,
        grid_spec=grid_spec,
        out_shape=jax.ShapeDtypeStruct((B, S + 1, D), feature.dtype),
        compiler_params=pltpu.CompilerParams(vmem_limit_bytes=100 * 1024 * 1024),
    )(idx, W, feature)


# speedup vs baseline: 21.3860x; 1.0004x over previous
import jax
import jax.numpy as jnp
from jax.experimental import pallas as pl
from jax.experimental.pallas import tpu as pltpu

_BB = 4
_SC = 2


def _concat_body(idx_ref, w_ref, feat_ref, out_ref):
    g = pl.program_id(0)
    c = pl.program_id(1)
    chunk = feat_ref.shape[1]

    @pl.when(c == 0)
    def _emb_rows():
        for j in range(_BB):
            out_ref[j, 0, :] = w_ref[idx_ref[g * _BB + j], :]

    for cc in range(_SC):

        @pl.when(c == cc)
        def _store_chunk(cc=cc):
            for j in range(_BB):
                out_ref[j, pl.ds(1 + cc * chunk, chunk), :] = feat_ref[j]


def kernel(feature, index_value, embedding_offset, W):
    B, S, D = feature.shape
    if isinstance(embedding_offset, int) and embedding_offset == 0:
        idx = index_value.astype(jnp.int32)
    else:
        idx = (index_value - embedding_offset).astype(jnp.int32)
    V = W.shape[0]
    grid_spec = pltpu.PrefetchScalarGridSpec(
        num_scalar_prefetch=1,
        grid=(B // _BB, _SC),
        in_specs=[
            pl.BlockSpec((V, D), lambda g, c, idx_ref: (0, 0)),
            pl.BlockSpec((_BB, S // _SC, D), lambda g, c, idx_ref: (g, c, 0)),
        ],
        out_specs=pl.BlockSpec((_BB, S + 1, D), lambda g, c, idx_ref: (g, 0, 0)),
    )
    return pl.pallas_call(
        _concat_body,
        grid_spec=grid_spec,
        out_shape=jax.ShapeDtypeStruct((B, S + 1, D), feature.dtype),
        compiler_params=pltpu.CompilerParams(vmem_limit_bytes=100 * 1024 * 1024),
    )(idx, W, feature)
